# baseline (speedup 1.0000x reference)
import jax
import jax.numpy as jnp
from jax import lax
from jax.experimental import pallas as pl
from jax.experimental.pallas import tpu as pltpu
from jax.experimental.pallas import tpu_sc as plsc

N = 256
HID = 128
EMB = 64
NCELL = 16
GW = NCELL * HID
NPRED = 12
NEG = -1e30
NHC = HID // 16

_NUM_WORKERS = 32
_ROWS = N // _NUM_WORKERS
_GSLOT = _ROWS * NCELL * 16


def _bins_from_rowcol(x_row, x_col, y_row, y_col):
    relx = x_col - x_row
    rely = y_col - y_row
    gx = jnp.minimum(jnp.maximum((relx + 1.0) * 2.0, 0.0), 3.0).astype(jnp.int32)
    gy = jnp.minimum(jnp.maximum((rely + 1.0) * 2.0, 0.0), 3.0).astype(jnp.int32)
    ri = lax.broadcasted_iota(jnp.int32, (N, N), 0)
    ci = lax.broadcasted_iota(jnp.int32, (N, N), 1)
    m = (jnp.abs(relx) <= 1.0) & (jnp.abs(rely) <= 1.0) & (ri != ci)
    invalid = 128 - NCELL * (ci & (_ROWS - 1))
    return jnp.where(m, gx * 4 + gy, invalid)


def _sc_grid_body(binsT_hbm, hid_hbm,
                  o0, o1, o2, o3, o4, o5, o6, o7,
                  binsT_v, hid_v,
                  g0, g1, g2, g3, g4, g5, g6, g7, dsem):
    wid = lax.axis_index("s") * 2 + lax.axis_index("c")
    outs = (o0, o1, o2, o3, o4, o5, o6, o7)
    grids = (g0, g1, g2, g3, g4, g5, g6, g7)
    cp_bins = pltpu.async_copy(binsT_hbm.at[pl.ds(wid, 1)], binsT_v, dsem)
    cp_hid = pltpu.async_copy(hid_hbm, hid_v, dsem)

    neg = jnp.full((16,), NEG, jnp.float32)

    def init_body(k, carry):
        for hc in range(NHC):
            grids[hc][pl.ds(k * 16, 16)] = neg
        return carry

    lax.fori_loop(0, _GSLOT // 16, init_body, 0)
    cp_bins.wait()
    cp_hid.wait()

    def j2_body(t, carry):
        for u in range(2):
            j2 = t * 2 + u
            bv = binsT_v[0, j2, :]
            for half in range(2):
                j = j2 * 2 + half
                hvs = [hid_v[j, pl.ds(hc * 16, 16)] for hc in range(NHC)]
                for a in range(_ROWS):
                    b = bv[half * _ROWS + a]
                    off = a * 256 + b * 16
                    for hc in range(NHC):
                        g = grids[hc][pl.ds(off, 16)]
                        grids[hc][pl.ds(off, 16)] = jnp.maximum(g, hvs[hc])
        return carry

    lax.fori_loop(0, N // 4, j2_body, 0)

    cps = [pltpu.async_copy(grids[hc].at[pl.ds(0, _GSLOT)], outs[hc].at[wid],
                            dsem)
           for hc in range(NHC)]
    for cp in cps:
        cp.wait()


_sc_grid = pl.kernel(
    _sc_grid_body,
    mesh=plsc.VectorSubcoreMesh(core_axis_name="c", subcore_axis_name="s"),
    out_type=[jax.ShapeDtypeStruct((_NUM_WORKERS, _GSLOT), jnp.float32)
              for _ in range(NHC)],
    scratch_types=[
        pltpu.VMEM((1, N // 2, 16), jnp.int32),
        pltpu.VMEM((N, HID), jnp.float32),
    ] + [pltpu.VMEM((_GSLOT + 16,), jnp.float32)] * NHC
      + [pltpu.SemaphoreType.DMA],
)


def _tc_step_body(pos_ref, prev_ref, x0, x1, x2, x3, x4, x5, x6, x7,
                  h_ref, c_ref,
                  wpos_t, bpos, wp0, wp1, wp2, wp3, wp4, wp5, wp6, wp7, bpool,
                  wih_emb_t, wih_soc_t, whh_t, bsum,
                  wout_t, bout,
                  h_out, c_out, pos_out, bins_out):
    xs = (x0, x1, x2, x3, x4, x5, x6, x7)
    wps = (wp0, wp1, wp2, wp3, wp4, wp5, wp6, wp7)
    pos = pos_ref[...]
    vel = pos - prev_ref[...]
    emb = jnp.maximum(
        jnp.dot(vel, wpos_t[...], preferred_element_type=jnp.float32) + bpos[...], 0.0)
    acc = bpool[...]
    for hc in range(NHC):
        g = xs[hc][...]
        g = jnp.where(g < -1e29, 0.0, g)
        acc = acc + jnp.dot(g, wps[hc][...], preferred_element_type=jnp.float32)
    social = jnp.maximum(acc, 0.0)
    gates = (jnp.dot(emb, wih_emb_t[...], preferred_element_type=jnp.float32)
             + jnp.dot(social, wih_soc_t[...], preferred_element_type=jnp.float32)
             + jnp.dot(h_ref[...], whh_t[...], preferred_element_type=jnp.float32)
             + bsum[...])
    gi = jax.nn.sigmoid(gates[:, 0:HID])
    gf = jax.nn.sigmoid(gates[:, HID:2 * HID])
    gg = jnp.tanh(gates[:, 2 * HID:3 * HID])
    go = jax.nn.sigmoid(gates[:, 3 * HID:4 * HID])
    c2 = gf * c_ref[...] + gi * gg
    h2 = go * jnp.tanh(c2)
    delta = jnp.dot(h2, wout_t[...], preferred_element_type=jnp.float32) + bout[...]
    nxt = pos + delta
    h_out[...] = h2
    c_out[...] = c2
    pos_out[...] = nxt

    eye = (lax.broadcasted_iota(jnp.int32, (N, N), 0)
           == lax.broadcasted_iota(jnp.int32, (N, N), 1)).astype(jnp.float32)
    x_col = nxt[:, 0:1]
    y_col = nxt[:, 1:2]
    x_row = jnp.sum(x_col * eye, axis=0, keepdims=True)
    y_row = jnp.sum(y_col * eye, axis=0, keepdims=True)
    bins_out[...] = _bins_from_rowcol(x_row, x_col, y_row, y_col)


_tc_step = pl.pallas_call(
    _tc_step_body,
    out_shape=[
        jax.ShapeDtypeStruct((N, HID), jnp.float32),
        jax.ShapeDtypeStruct((N, HID), jnp.float32),
        jax.ShapeDtypeStruct((N, 2), jnp.float32),
        jax.ShapeDtypeStruct((N, N), jnp.int32),
    ],
)


def _tc_bins_enc_body(ox_ref, oxT_ref, oy_ref, oyT_ref, out_ref):
    for t in range(1, 8):
        x_row = ox_ref[t:t + 1, :]
        y_row = oy_ref[t:t + 1, :]
        x_col = oxT_ref[:, t:t + 1]
        y_col = oyT_ref[:, t:t + 1]
        out_ref[t - 1] = _bins_from_rowcol(x_row, x_col, y_row, y_col)


_tc_bins_enc = pl.pallas_call(
    _tc_bins_enc_body,
    out_shape=jax.ShapeDtypeStruct((7, N, N), jnp.int32),
)


def kernel(observed, n_predict, W_pos, b_pos, W_pool, b_pool,
           W_ih_e, W_hh_e, b_ih_e, b_hh_e,
           W_ih_d, W_hh_d, b_ih_d, b_hh_d, W_out, b_out):
    obs = observed

    wpos_t = W_pos.T
    wpool_t = W_pool.T.reshape(NCELL, NHC, 16, HID)
    wps = [wpool_t[:, hc].reshape(NCELL * 16, HID) for hc in range(NHC)]
    enc = (W_ih_e[:, :EMB].T, W_ih_e[:, EMB:].T, W_hh_e.T,
           (b_ih_e + b_hh_e).reshape(1, -1))
    dec = (W_ih_d[:, :EMB].T, W_ih_d[:, EMB:].T, W_hh_d.T,
           (b_ih_d + b_hh_d).reshape(1, -1))
    wout_t = W_out.T
    bpos = b_pos.reshape(1, -1)
    bpool = b_pool.reshape(1, -1)
    bout = b_out.reshape(1, -1)

    ox = obs[:, :, 0]
    oy = obs[:, :, 1]
    bins_enc = _tc_bins_enc(ox, ox.T, oy, oy.T)

    def step(cell_p, h, c, pos, prev, bins):
        wih_emb_t, wih_soc_t, whh_t, bsum = cell_p
        binsT = bins.reshape(N // 2, 2, _NUM_WORKERS, _ROWS).transpose(
            2, 0, 1, 3).reshape(_NUM_WORKERS, N // 2, 16)
        gouts = _sc_grid(binsT, h)
        xs = [g.reshape(N, NCELL * 16) for g in gouts]
        return _tc_step(pos, prev, *xs, h, c,
                        wpos_t, bpos, *wps, bpool,
                        wih_emb_t, wih_soc_t, whh_t, bsum,
                        wout_t, bout)

    h = jnp.zeros((N, HID), jnp.float32)
    c = jnp.zeros((N, HID), jnp.float32)
    for t in range(1, 8):
        h, c, _, _ = step(enc, h, c, obs[t], obs[t - 1], bins_enc[t - 1])
    preds = []
    prev = obs[7]
    curr = obs[7]
    bins = bins_enc[6]
    for _ in range(NPRED):
        h, c, nxt, bins_nxt = step(dec, h, c, curr, prev, bins)
        preds.append(nxt)
        prev, curr = curr, nxt
        bins = bins_nxt
    return jnp.stack(preds)


# speedup vs baseline: 1.0145x; 1.0145x over previous
import jax
import jax.numpy as jnp
from jax import lax
from jax.experimental import pallas as pl
from jax.experimental.pallas import tpu as pltpu
from jax.experimental.pallas import tpu_sc as plsc

N = 256
HID = 128
EMB = 64
NCELL = 16
GW = NCELL * HID
NPRED = 12
NEG = -1e30
NHC = HID // 16

_NUM_WORKERS = 32
_ROWS = N // _NUM_WORKERS
_GSLOT = _ROWS * NCELL * 16


def _bins_from_rowcol(x_row, x_col, y_row, y_col):
    relx = x_col - x_row
    rely = y_col - y_row
    gx = jnp.minimum(jnp.maximum((relx + 1.0) * 2.0, 0.0), 3.0).astype(jnp.int32)
    gy = jnp.minimum(jnp.maximum((rely + 1.0) * 2.0, 0.0), 3.0).astype(jnp.int32)
    ri = lax.broadcasted_iota(jnp.int32, (N, N), 0)
    ci = lax.broadcasted_iota(jnp.int32, (N, N), 1)
    m = (jnp.abs(relx) <= 1.0) & (jnp.abs(rely) <= 1.0) & (ri != ci)
    invalid = 128 - NCELL * (ci & (_ROWS - 1))
    return jnp.where(m, gx * 4 + gy, invalid)


def _sc_grid_body(binsT_hbm, hid_hbm,
                  o0, o1, o2, o3, o4, o5, o6, o7,
                  binsT_v, hid_v,
                  g0, g1, g2, g3, g4, g5, g6, g7, dsem):
    wid = lax.axis_index("s") * 2 + lax.axis_index("c")
    outs = (o0, o1, o2, o3, o4, o5, o6, o7)
    grids = (g0, g1, g2, g3, g4, g5, g6, g7)
    cp_bins = pltpu.async_copy(binsT_hbm.at[pl.ds(wid, 1)], binsT_v, dsem)
    cp_hid = pltpu.async_copy(hid_hbm, hid_v, dsem)

    neg = jnp.full((16,), NEG, jnp.float32)

    def init_body(k, carry):
        for hc in range(NHC):
            grids[hc][pl.ds(k * 16, 16)] = neg
        return carry

    lax.fori_loop(0, _GSLOT // 16, init_body, 0)
    cp_bins.wait()
    cp_hid.wait()

    def j2_body(j2, carry):
        bv = binsT_v[0, j2, :]
        for half in range(2):
            j = j2 * 2 + half
            hvs = [hid_v[j, pl.ds(hc * 16, 16)] for hc in range(NHC)]
            for a in range(_ROWS):
                b = bv[half * _ROWS + a]
                off = a * 256 + b * 16
                for hc in range(NHC):
                    g = grids[hc][pl.ds(off, 16)]
                    grids[hc][pl.ds(off, 16)] = jnp.maximum(g, hvs[hc])
        return carry

    lax.fori_loop(0, N // 2, j2_body, 0)

    cps = [pltpu.async_copy(grids[hc].at[pl.ds(0, _GSLOT)], outs[hc].at[wid],
                            dsem)
           for hc in range(NHC)]
    for cp in cps:
        cp.wait()


_sc_grid = pl.kernel(
    _sc_grid_body,
    mesh=plsc.VectorSubcoreMesh(core_axis_name="c", subcore_axis_name="s"),
    out_type=[jax.ShapeDtypeStruct((_NUM_WORKERS, _GSLOT), jnp.float32)
              for _ in range(NHC)],
    scratch_types=[
        pltpu.VMEM((1, N // 2, 16), jnp.int32),
        pltpu.VMEM((N, HID), jnp.float32),
    ] + [pltpu.VMEM((_GSLOT + 16,), jnp.float32)] * NHC
      + [pltpu.SemaphoreType.DMA],
)


def _tc_step_body(pos_ref, prev_ref, x0, x1, x2, x3, x4, x5, x6, x7,
                  h_ref, c_ref,
                  wpos_t, bpos, wp0, wp1, wp2, wp3, wp4, wp5, wp6, wp7, bpool,
                  wih_emb_t, wih_soc_t, whh_t, bsum,
                  wout_t, bout,
                  h_out, c_out, pos_out, bins_out):
    xs = (x0, x1, x2, x3, x4, x5, x6, x7)
    wps = (wp0, wp1, wp2, wp3, wp4, wp5, wp6, wp7)
    pos = pos_ref[...]
    vel = pos - prev_ref[...]
    emb = jnp.maximum(
        jnp.dot(vel, wpos_t[...], preferred_element_type=jnp.float32) + bpos[...], 0.0)
    acc = bpool[...]
    for hc in range(NHC):
        g = xs[hc][...]
        g = jnp.where(g < -1e29, 0.0, g)
        acc = acc + jnp.dot(g, wps[hc][...], preferred_element_type=jnp.float32)
    social = jnp.maximum(acc, 0.0)
    gates = (jnp.dot(emb, wih_emb_t[...], preferred_element_type=jnp.float32)
             + jnp.dot(social, wih_soc_t[...], preferred_element_type=jnp.float32)
             + jnp.dot(h_ref[...], whh_t[...], preferred_element_type=jnp.float32)
             + bsum[...])
    gi = jax.nn.sigmoid(gates[:, 0:HID])
    gf = jax.nn.sigmoid(gates[:, HID:2 * HID])
    gg = jnp.tanh(gates[:, 2 * HID:3 * HID])
    go = jax.nn.sigmoid(gates[:, 3 * HID:4 * HID])
    c2 = gf * c_ref[...] + gi * gg
    h2 = go * jnp.tanh(c2)
    delta = jnp.dot(h2, wout_t[...], preferred_element_type=jnp.float32) + bout[...]
    nxt = pos + delta
    h_out[...] = h2
    c_out[...] = c2
    pos_out[...] = nxt

    eye = (lax.broadcasted_iota(jnp.int32, (N, N), 0)
           == lax.broadcasted_iota(jnp.int32, (N, N), 1)).astype(jnp.float32)
    x_col = nxt[:, 0:1]
    y_col = nxt[:, 1:2]
    x_row = jnp.sum(x_col * eye, axis=0, keepdims=True)
    y_row = jnp.sum(y_col * eye, axis=0, keepdims=True)
    bins_out[...] = _bins_from_rowcol(x_row, x_col, y_row, y_col)


_tc_step = pl.pallas_call(
    _tc_step_body,
    out_shape=[
        jax.ShapeDtypeStruct((N, HID), jnp.float32),
        jax.ShapeDtypeStruct((N, HID), jnp.float32),
        jax.ShapeDtypeStruct((N, 2), jnp.float32),
        jax.ShapeDtypeStruct((N, N), jnp.int32),
    ],
)


def _tc_bins_enc_body(ox_ref, oxT_ref, oy_ref, oyT_ref, out_ref):
    for t in range(1, 8):
        x_row = ox_ref[t:t + 1, :]
        y_row = oy_ref[t:t + 1, :]
        x_col = oxT_ref[:, t:t + 1]
        y_col = oyT_ref[:, t:t + 1]
        out_ref[t - 1] = _bins_from_rowcol(x_row, x_col, y_row, y_col)


_tc_bins_enc = pl.pallas_call(
    _tc_bins_enc_body,
    out_shape=jax.ShapeDtypeStruct((7, N, N), jnp.int32),
)


def kernel(observed, n_predict, W_pos, b_pos, W_pool, b_pool,
           W_ih_e, W_hh_e, b_ih_e, b_hh_e,
           W_ih_d, W_hh_d, b_ih_d, b_hh_d, W_out, b_out):
    obs = observed

    wpos_t = W_pos.T
    wpool_t = W_pool.T.reshape(NCELL, NHC, 16, HID)
    wps = [wpool_t[:, hc].reshape(NCELL * 16, HID) for hc in range(NHC)]
    enc = (W_ih_e[:, :EMB].T, W_ih_e[:, EMB:].T, W_hh_e.T,
           (b_ih_e + b_hh_e).reshape(1, -1))
    dec = (W_ih_d[:, :EMB].T, W_ih_d[:, EMB:].T, W_hh_d.T,
           (b_ih_d + b_hh_d).reshape(1, -1))
    wout_t = W_out.T
    bpos = b_pos.reshape(1, -1)
    bpool = b_pool.reshape(1, -1)
    bout = b_out.reshape(1, -1)

    ox = obs[:, :, 0]
    oy = obs[:, :, 1]
    bins_enc = _tc_bins_enc(ox, ox.T, oy, oy.T)

    def step(cell_p, h, c, pos, prev, bins):
        wih_emb_t, wih_soc_t, whh_t, bsum = cell_p
        binsT = bins.reshape(N // 2, 2, _NUM_WORKERS, _ROWS).transpose(
            2, 0, 1, 3).reshape(_NUM_WORKERS, N // 2, 16)
        gouts = _sc_grid(binsT, h)
        xs = [g.reshape(N, NCELL * 16) for g in gouts]
        return _tc_step(pos, prev, *xs, h, c,
                        wpos_t, bpos, *wps, bpool,
                        wih_emb_t, wih_soc_t, whh_t, bsum,
                        wout_t, bout)

    h = jnp.zeros((N, HID), jnp.float32)
    c = jnp.zeros((N, HID), jnp.float32)
    for t in range(1, 8):
        h, c, _, _ = step(enc, h, c, obs[t], obs[t - 1], bins_enc[t - 1])
    preds = []
    prev = obs[7]
    curr = obs[7]
    bins = bins_enc[6]
    for _ in range(NPRED):
        h, c, nxt, bins_nxt = step(dec, h, c, curr, prev, bins)
        preds.append(nxt)
        prev, curr = curr, nxt
        bins = bins_nxt
    return jnp.stack(preds)
